# baseline (speedup 1.0000x reference)
import functools

import jax
import jax.numpy as jnp
from jax import lax
from jax.experimental import pallas as pl
from jax.experimental.pallas import tpu as pltpu
from jax.experimental.pallas import tpu_sc as plsc

NC = 2
NS = 16
NW = NC * NS
LN = 16
NPAD = 10240
SLICE = NPAD // NS
EB = 128
ROWS_TOTAL = 2500
ROWS_BASE = ROWS_TOTAL // NW
ROWS_EXTRA = ROWS_TOTAL - ROWS_BASE * NW
RPT = ROWS_BASE + 1
NIT = (RPT + 1) // 2
WID = 16
G = 64
PBLK = 1000
NBLK = 10

_SC_PARAMS = dict(
    compiler_params=pltpu.CompilerParams(
        needs_layout_passes=False, use_tc_tiling_on_sc=False))


def _mesh():
    return plsc.VectorSubcoreMesh(
        core_axis_name="c", subcore_axis_name="s",
        num_cores=NC, num_subcores=NS)


def _wid_rows():
    cid = lax.axis_index("c")
    sid = lax.axis_index("s")
    wid = sid * NC + cid
    r0 = wid * ROWS_BASE + jnp.minimum(wid, ROWS_EXTRA)
    cnt = ROWS_BASE + jnp.where(wid < ROWS_EXTRA, 1, 0)
    return cid, sid, r0, cnt


def _zero_1d(ref, n):
    def z(i, carry):
        ref[pl.ds(i * LN, LN)] = jnp.zeros((LN,), jnp.float32)
        return carry
    lax.fori_loop(0, n // LN, z, 0)


def _make_shufxor():
    keys = {sh: jnp.arange(LN, dtype=jnp.int32) ^ sh for sh in (1, 2, 4, 8)}

    def shufxor(x, sh):
        _, out = plsc.sort_key_val(keys[sh], x)
        return out
    return shufxor


def _butterfly(regs, lane, shufxor):
    sh = 1
    while len(regs) > 1:
        msk = (lane & sh) == 0
        nxt = []
        for i2 in range(0, len(regs), 2):
            a, b = regs[i2], regs[i2 + 1]
            c = (jnp.where(msk, a, b)
                 + shufxor(jnp.where(msk, b, a), sh))
            nxt.append(c)
        regs = nxt
        sh *= 2
    return regs[0]


def _edge_pass_a():
    def body(xl_hbm, xr_hbm, ep_hbm, ewb_hbm, attb_hbm,
             ex_out, den_out,
             eslab, exslab, rows_s0, rows_d0, rows_s1, rows_d1, ew_sv,
             att_sv, den_v, mrg_a, mrg_b, abuf, den_sh,
             sem_s0, sem_d0, sem_s1, sem_d1):
        cid, sid, r0, cnt = _wid_rows()
        rs = jnp.minimum(r0, ROWS_TOTAL - RPT)
        off = r0 - rs
        pltpu.sync_copy(ep_hbm.at[pl.ds(rs, RPT)], eslab)
        pltpu.sync_copy(ewb_hbm, ew_sv)
        pltpu.sync_copy(attb_hbm, att_sv)
        ew_vec = ew_sv[pl.ds(0, LN)]
        att_vec = att_sv[pl.ds(0, LN)]
        _zero_1d(den_v, NPAD)
        lane = lax.iota(jnp.int32, LN)
        shufxor = _make_shufxor()

        def slabrow(i):
            return jnp.minimum(off + i, RPT - 1)

        def start_gathers(j, rs_buf, rd_buf, ss, sd):
            pltpu.async_copy(xl_hbm.at[eslab.at[j, 0]], rs_buf, ss)
            pltpu.async_copy(xr_hbm.at[eslab.at[j, 1]], rd_buf, sd)

        def wait_gather(buf, sem):
            pltpu.make_async_copy(xl_hbm.at[pl.ds(0, EB)], buf, sem).wait()

        def compute_row(i, rows_s, rows_d):
            row = slabrow(i)
            valid = i < cnt
            for v in range(EB // LN):
                d16 = eslab[row, 1, pl.ds(v * LN, LN)]
                attr16 = plsc.bitcast(
                    eslab[row, 2, pl.ds(v * LN, LN)], jnp.float32)
                abuf[pl.ds(v * LN, LN)] = jnp.zeros((LN,), jnp.float32)
                for j in range(LN):
                    e = v * LN + j
                    u = rows_s[e, :] + rows_d[e, :] + attr16[j] * ew_vec
                    m = jnp.maximum(u, 0.2 * u)
                    plsc.addupdate_scatter(
                        abuf, [jnp.full((LN,), e, jnp.int32)], att_vec * m)
                alpha16 = abuf[pl.ds(v * LN, LN)]
                e16 = jnp.where(valid, jnp.exp(alpha16), 0.0)
                exslab[i, pl.ds(v * LN, LN)] = e16
                plsc.addupdate_scatter(den_v, [d16], e16)

        start_gathers(slabrow(0), rows_s0, rows_d0, sem_s0, sem_d0)
        start_gathers(slabrow(1), rows_s1, rows_d1, sem_s1, sem_d1)

        def it(k, carry):
            i0 = 2 * k
            i1 = 2 * k + 1
            wait_gather(rows_s0, sem_s0)
            wait_gather(rows_d0, sem_d0)
            compute_row(i0, rows_s0, rows_d0)

            @pl.when(k < NIT - 1)
            def _():
                start_gathers(slabrow(i0 + 2), rows_s0, rows_d0,
                              sem_s0, sem_d0)
            wait_gather(rows_s1, sem_s1)
            wait_gather(rows_d1, sem_d1)
            compute_row(i1, rows_s1, rows_d1)

            @pl.when(k < NIT - 1)
            def _():
                start_gathers(slabrow(i1 + 2), rows_s1, rows_d1,
                              sem_s1, sem_d1)
            return carry
        lax.fori_loop(0, NIT, it, 0)

        pltpu.sync_copy(exslab.at[pl.ds(0, ROWS_BASE)],
                        ex_out.at[pl.ds(r0, ROWS_BASE)])

        @pl.when(cnt > ROWS_BASE)
        def _():
            pltpu.sync_copy(exslab.at[ROWS_BASE],
                            ex_out.at[r0 + ROWS_BASE])

        pltpu.sync_copy(den_v, den_sh.at[sid])
        plsc.subcore_barrier()
        o2 = sid * SLICE
        pltpu.sync_copy(den_sh.at[0, pl.ds(o2, SLICE)], mrg_a)
        for t in range(1, NS):
            pltpu.sync_copy(den_sh.at[t, pl.ds(o2, SLICE)], mrg_b)

            def addv(i, carry):
                mrg_a[pl.ds(i * LN, LN)] = (
                    mrg_a[pl.ds(i * LN, LN)] + mrg_b[pl.ds(i * LN, LN)])
                return carry
            lax.fori_loop(0, SLICE // LN, addv, 0)
        pltpu.sync_copy(mrg_a, den_out.at[cid, pl.ds(o2, SLICE)])

    return pl.kernel(
        body,
        out_type=[jax.ShapeDtypeStruct((ROWS_TOTAL, EB), jnp.float32),
                  jax.ShapeDtypeStruct((NC, NPAD), jnp.float32)],
        mesh=_mesh(),
        scratch_types=[
            pltpu.VMEM((RPT, 3, EB), jnp.int32),
            pltpu.VMEM((RPT + 1, EB), jnp.float32),
            pltpu.VMEM((EB, WID), jnp.float32),
            pltpu.VMEM((EB, WID), jnp.float32),
            pltpu.VMEM((EB, WID), jnp.float32),
            pltpu.VMEM((EB, WID), jnp.float32),
            pltpu.VMEM((LN,), jnp.float32),
            pltpu.VMEM((LN,), jnp.float32),
            pltpu.VMEM((NPAD,), jnp.float32),
            pltpu.VMEM((SLICE,), jnp.float32),
            pltpu.VMEM((SLICE,), jnp.float32),
            pltpu.VMEM((EB,), jnp.float32),
            pltpu.VMEM_SHARED((NS, NPAD), jnp.float32),
            pltpu.SemaphoreType.DMA,
            pltpu.SemaphoreType.DMA,
            pltpu.SemaphoreType.DMA,
            pltpu.SemaphoreType.DMA,
        ],
        **_SC_PARAMS)


def _edge_pass_b():
    def body(xl_hbm, ep_hbm, ex_hbm, den_hbm, zeros_hbm,
             out_p,
             eslab, exslab, rows_s0, rows_s1, scaled0, scaled1,
             den_v, den_v2, out_sh,
             sem_s0, sem_s1, sem_o0, sem_o1):
        cid, sid, r0, cnt = _wid_rows()
        rs = jnp.minimum(r0, ROWS_TOTAL - RPT)
        off = r0 - rs
        pltpu.sync_copy(ep_hbm.at[pl.ds(rs, RPT)], eslab)
        pltpu.sync_copy(ex_hbm.at[pl.ds(rs, RPT)], exslab)
        pltpu.sync_copy(den_hbm.at[0], den_v)
        pltpu.sync_copy(den_hbm.at[1], den_v2)

        def addv(i, carry):
            den_v[pl.ds(i * LN, LN)] = (
                den_v[pl.ds(i * LN, LN)] + den_v2[pl.ds(i * LN, LN)])
            return carry
        lax.fori_loop(0, NPAD // LN, addv, 0)

        o2 = sid * SLICE
        pltpu.sync_copy(zeros_hbm.at[pl.ds(o2, SLICE), :],
                        out_sh.at[pl.ds(o2, SLICE), :])
        plsc.subcore_barrier()

        def slabrow(i):
            return jnp.minimum(off + i, RPT - 1)

        def start_gather(j, rs_buf, ss):
            pltpu.async_copy(xl_hbm.at[eslab.at[j, 0]], rs_buf, ss)

        def wait_gather(buf, sem):
            pltpu.make_async_copy(xl_hbm.at[pl.ds(0, EB)], buf, sem).wait()

        def wait_scatter(buf, j, sem):
            pltpu.make_async_copy(buf, out_sh.at[eslab.at[j, 1]], sem).wait()

        def compute_scaled(i, rows_s, scaled):
            row = slabrow(i)
            valid = i < cnt
            for v in range(EB // LN):
                d16 = eslab[row, 1, pl.ds(v * LN, LN)]
                e16 = exslab[row, pl.ds(v * LN, LN)]
                den16 = plsc.load_gather(den_v, [d16])
                a16 = jnp.where(valid, e16 / (den16 + 1e-16), 0.0)
                for j in range(LN):
                    e = v * LN + j
                    scaled[e, :] = a16[j] * rows_s[e, :]

        start_gather(slabrow(0), rows_s0, sem_s0)
        start_gather(slabrow(1), rows_s1, sem_s1)

        def it(k, carry):
            i0 = 2 * k
            i1 = 2 * k + 1
            wait_gather(rows_s0, sem_s0)

            @pl.when(k > 0)
            def _():
                wait_scatter(scaled0, slabrow(i0 - 2), sem_o0)
            compute_scaled(i0, rows_s0, scaled0)
            pltpu.async_copy(scaled0, out_sh.at[eslab.at[slabrow(i0), 1]],
                             sem_o0, add=True)

            @pl.when(k < NIT - 1)
            def _():
                start_gather(slabrow(i0 + 2), rows_s0, sem_s0)
            wait_gather(rows_s1, sem_s1)

            @pl.when(k > 0)
            def _():
                wait_scatter(scaled1, slabrow(i1 - 2), sem_o1)
            compute_scaled(i1, rows_s1, scaled1)
            pltpu.async_copy(scaled1, out_sh.at[eslab.at[slabrow(i1), 1]],
                             sem_o1, add=True)

            @pl.when(k < NIT - 1)
            def _():
                start_gather(slabrow(i1 + 2), rows_s1, sem_s1)
            return carry
        lax.fori_loop(0, NIT, it, 0)
        wait_scatter(scaled0, slabrow(2 * NIT - 2), sem_o0)
        wait_scatter(scaled1, slabrow(2 * NIT - 1), sem_o1)
        plsc.subcore_barrier()
        pltpu.sync_copy(out_sh.at[pl.ds(o2, SLICE), :],
                        out_p.at[cid, pl.ds(o2, SLICE), :])

    return pl.kernel(
        body,
        out_type=[jax.ShapeDtypeStruct((NC, NPAD, WID), jnp.float32)],
        mesh=_mesh(),
        scratch_types=[
            pltpu.VMEM((RPT, 3, EB), jnp.int32),
            pltpu.VMEM((RPT, EB), jnp.float32),
            pltpu.VMEM((EB, WID), jnp.float32),
            pltpu.VMEM((EB, WID), jnp.float32),
            pltpu.VMEM((EB, WID), jnp.float32),
            pltpu.VMEM((EB, WID), jnp.float32),
            pltpu.VMEM((NPAD,), jnp.float32),
            pltpu.VMEM((NPAD,), jnp.float32),
            pltpu.VMEM_SHARED((NPAD, WID), jnp.float32),
            pltpu.SemaphoreType.DMA,
            pltpu.SemaphoreType.DMA,
            pltpu.SemaphoreType.DMA,
            pltpu.SemaphoreType.DMA,
        ],
        **_SC_PARAMS)


def _proj_kernel(x_ref, wl_ref, bl_ref, wr_ref, br_ref, xl_ref, xr_ref):
    xv = x_ref[...]
    xl_ref[...] = jnp.dot(xv, wl_ref[...],
                          preferred_element_type=jnp.float32) + bl_ref[...]
    xr_ref[...] = jnp.dot(xv, wr_ref[...],
                          preferred_element_type=jnp.float32) + br_ref[...]


def _proj(x, wl, bl, wr, br):
    n, d = x.shape
    k = wl.shape[1]
    blk = 1024
    return pl.pallas_call(
        _proj_kernel,
        grid=(n // blk,),
        in_specs=[
            pl.BlockSpec((blk, d), lambda i: (i, 0)),
            pl.BlockSpec((d, k), lambda i: (0, 0)),
            pl.BlockSpec((1, k), lambda i: (0, 0)),
            pl.BlockSpec((d, k), lambda i: (0, 0)),
            pl.BlockSpec((1, k), lambda i: (0, 0)),
        ],
        out_specs=[pl.BlockSpec((blk, k), lambda i: (i, 0)),
                   pl.BlockSpec((blk, k), lambda i: (i, 0))],
        out_shape=[jax.ShapeDtypeStruct((n, k), jnp.float32)] * 2,
    )(x, wl, bl, wr, br)


def _merge_proj_kernel(hin, p_ref, b1_ref, wl_ref, bl_ref, wr_ref, br_ref,
                       xl_ref, xr_ref):
    p = p_ref[0][:, :hin] + p_ref[1][:, :hin]
    h = jnp.maximum(p + b1_ref[...], 0.0)
    xl_ref[...] = jnp.dot(h, wl_ref[...],
                          preferred_element_type=jnp.float32) + bl_ref[...]
    xr_ref[...] = jnp.dot(h, wr_ref[...],
                          preferred_element_type=jnp.float32) + br_ref[...]


def _merge_proj(p, b1, wl, bl, wr, br):
    _, n, wid = p.shape
    hin = wl.shape[0]
    k = wl.shape[1]
    blk = 1024
    return pl.pallas_call(
        functools.partial(_merge_proj_kernel, hin),
        grid=(n // blk,),
        in_specs=[
            pl.BlockSpec((NC, blk, wid), lambda i: (0, i, 0)),
            pl.BlockSpec((1, hin), lambda i: (0, 0)),
            pl.BlockSpec((hin, k), lambda i: (0, 0)),
            pl.BlockSpec((1, k), lambda i: (0, 0)),
            pl.BlockSpec((hin, k), lambda i: (0, 0)),
            pl.BlockSpec((1, k), lambda i: (0, 0)),
        ],
        out_specs=[pl.BlockSpec((blk, k), lambda i: (i, 0)),
                   pl.BlockSpec((blk, k), lambda i: (i, 0))],
        out_shape=[jax.ShapeDtypeStruct((n, k), jnp.float32)] * 2,
    )(p, b1, wl, bl, wr, br)


def _pool_kernel(p_ref, b2_ref, batch_ref, w3t_ref, b3_ref, y_ref,
                 s_acc, c_acc):
    k = pl.program_id(0)

    @pl.when(k == 0)
    def _():
        s_acc[...] = jnp.zeros_like(s_acc)
        c_acc[...] = jnp.zeros_like(c_acc)

    h = jnp.maximum(p_ref[0] + p_ref[1] + b2_ref[...], 0.0)
    b = batch_ref[0, 0, :]
    gi = lax.broadcasted_iota(jnp.int32, (G, PBLK), 0)
    oh = (gi == b[None, :]).astype(jnp.float32)
    s_acc[...] += jnp.dot(oh, h, preferred_element_type=jnp.float32)
    c_acc[...] += jnp.sum(oh, axis=1, keepdims=True)

    @pl.when(k == NBLK - 1)
    def _():
        g = s_acc[...] / jnp.maximum(c_acc[...], 1.0)
        y_ref[...] = jnp.dot(g, w3t_ref[...],
                             preferred_element_type=jnp.float32) + b3_ref[...]


def _pool(p, b2, batch3, w3t, b3):
    hin = p.shape[2]
    return pl.pallas_call(
        _pool_kernel,
        grid=(NBLK,),
        in_specs=[
            pl.BlockSpec((NC, PBLK, hin), lambda i: (0, i, 0)),
            pl.BlockSpec((1, hin), lambda i: (0, 0)),
            pl.BlockSpec((1, 1, PBLK), lambda i: (i, 0, 0)),
            pl.BlockSpec((hin, 1), lambda i: (0, 0)),
            pl.BlockSpec((1, 1), lambda i: (0, 0)),
        ],
        out_specs=pl.BlockSpec((G, 1), lambda i: (0, 0)),
        out_shape=jax.ShapeDtypeStruct((G, 1), jnp.float32),
        scratch_shapes=[pltpu.VMEM((G, hin), jnp.float32),
                        pltpu.VMEM((G, 1), jnp.float32)],
    )(p, b2, batch3, w3t, b3)


def kernel(x, edge_index, edge_attr, batch, lin_l_w1, lin_l_b1, lin_r_w1,
           lin_r_b1, lin_edge_w1, att1, bias1, lin_l_w2, lin_l_b2, lin_r_w2,
           lin_r_b2, lin_edge_w2, att2, bias2, w3, b3):
    n, _ = x.shape
    h1 = lin_l_w1.shape[0]
    h2 = lin_l_w2.shape[0]

    src2d = edge_index[0].reshape(ROWS_TOTAL, EB)
    dst2d = edge_index[1].reshape(ROWS_TOTAL, EB)
    attr2d = lax.bitcast_convert_type(
        edge_attr.reshape(ROWS_TOTAL, EB), jnp.int32)
    epack = jnp.stack([src2d, dst2d, attr2d], axis=1)
    x_pad = jnp.pad(x, ((0, NPAD - n), (0, 0)))
    zeros_nw = jnp.zeros((NPAD, WID), jnp.float32)

    wl1 = jnp.pad(lin_l_w1.T, ((0, 0), (0, WID - h1)))
    wr1 = jnp.pad(lin_r_w1.T, ((0, 0), (0, WID - h1)))
    bl1 = jnp.pad(lin_l_b1, (0, WID - h1))[None]
    br1 = jnp.pad(lin_r_b1, (0, WID - h1))[None]
    xl1, xr1 = _proj(x_pad, wl1, bl1, wr1, br1)
    ewb1 = jnp.pad(lin_edge_w1[:, 0], (0, WID - h1))
    attb1 = jnp.pad(att1, (0, WID - h1))
    ex1, den1 = _edge_pass_a()(xl1, xr1, epack, ewb1, attb1)
    outp1, = _edge_pass_b()(xl1, epack, ex1, den1, zeros_nw)

    xl2, xr2 = _merge_proj(outp1, bias1[None], lin_l_w2.T, lin_l_b2[None],
                           lin_r_w2.T, lin_r_b2[None])
    ewb2 = jnp.pad(lin_edge_w2[:, 0], (0, WID - h2))
    attb2 = jnp.pad(att2, (0, WID - h2))
    ex2, den2 = _edge_pass_a()(xl2, xr2, epack, ewb2, attb2)
    outp2, = _edge_pass_b()(xl2, epack, ex2, den2, zeros_nw)

    y = _pool(outp2[:, :n, :], bias2[None], batch.reshape(NBLK, 1, PBLK),
              w3.T, b3[None])
    return y


# speedup vs baseline: 1.9601x; 1.9601x over previous
import functools

import jax
import jax.numpy as jnp
from jax import lax
from jax.experimental import pallas as pl
from jax.experimental.pallas import tpu as pltpu
from jax.experimental.pallas import tpu_sc as plsc

NC = 2
NS = 16
NW = NC * NS
LN = 16
NPAD = 10240
SLICE = NPAD // NS
EB = 128
ROWS_TOTAL = 2500
ROWS_BASE = ROWS_TOTAL // NW
ROWS_EXTRA = ROWS_TOTAL - ROWS_BASE * NW
RPT = ROWS_BASE + 1
NIT4 = (RPT + 1) // 4
WID = 16
G = 64
PBLK = 1000
NBLK = 10

_SC_PARAMS = dict(
    compiler_params=pltpu.CompilerParams(
        needs_layout_passes=False, use_tc_tiling_on_sc=False))


def _mesh():
    return plsc.VectorSubcoreMesh(
        core_axis_name="c", subcore_axis_name="s",
        num_cores=NC, num_subcores=NS)


def _wid_rows():
    cid = lax.axis_index("c")
    sid = lax.axis_index("s")
    wid = sid * NC + cid
    r0 = wid * ROWS_BASE + jnp.minimum(wid, ROWS_EXTRA)
    cnt = ROWS_BASE + jnp.where(wid < ROWS_EXTRA, 1, 0)
    return cid, sid, r0, cnt


def _zero_1d(ref, n):
    def z(i, carry):
        for q in range(4):
            ref[pl.ds(i * 4 * LN + q * LN, LN)] = jnp.zeros((LN,),
                                                            jnp.float32)
        return carry
    lax.fori_loop(0, n // (4 * LN), z, 0)


def _make_shufxor():
    keys = {sh: jnp.arange(LN, dtype=jnp.int32) ^ sh for sh in (1, 2, 4, 8)}

    def shufxor(x, sh):
        _, out = plsc.sort_key_val(keys[sh], x)
        return out
    return shufxor


def _butterfly(regs, lane, shufxor):
    sh = 1
    while len(regs) > 1:
        msk = (lane & sh) == 0
        nxt = []
        for i2 in range(0, len(regs), 2):
            a, b = regs[i2], regs[i2 + 1]
            c = (jnp.where(msk, a, b)
                 + shufxor(jnp.where(msk, b, a), sh))
            nxt.append(c)
        regs = nxt
        sh *= 2
    return regs[0]


def _edge_pass_a():
    def body(xl_hbm, xr_hbm, ep_hbm, ewb_hbm, attb_hbm,
             ex_out, den_out,
             eslab, exslab, rows_s0, rows_d0, rows_s1, rows_d1,
             rows_s2, rows_d2, rows_s3, rows_d3, ew_sv,
             att_sv, den_v, mrg_a, mrg_b, den_sh,
             sem_s0, sem_d0, sem_s1, sem_d1,
             sem_s2, sem_d2, sem_s3, sem_d3):
        cid, sid, r0, cnt = _wid_rows()
        rs = jnp.minimum(r0, ROWS_TOTAL - RPT)
        off = r0 - rs
        pltpu.sync_copy(ep_hbm.at[pl.ds(rs, RPT)], eslab)
        pltpu.sync_copy(ewb_hbm, ew_sv)
        pltpu.sync_copy(attb_hbm, att_sv)
        ew_vec = ew_sv[pl.ds(0, LN)]
        att_vec = att_sv[pl.ds(0, LN)]
        _zero_1d(den_v, NPAD)
        lane = lax.iota(jnp.int32, LN)
        shufxor = _make_shufxor()

        def slabrow(i):
            return jnp.minimum(off + i, RPT - 1)

        def start_gathers(j, rs_buf, rd_buf, ss, sd):
            pltpu.async_copy(xl_hbm.at[eslab.at[j, 0]], rs_buf, ss)
            pltpu.async_copy(xr_hbm.at[eslab.at[j, 1]], rd_buf, sd)

        def wait_gather(buf, sem):
            pltpu.make_async_copy(xl_hbm.at[pl.ds(0, EB)], buf, sem).wait()

        def compute_row(i, rows_s, rows_d):
            row = slabrow(i)
            valid = i < cnt
            for v in range(EB // LN):
                d16 = eslab[row, 1, pl.ds(v * LN, LN)]
                attr16 = plsc.bitcast(
                    eslab[row, 2, pl.ds(v * LN, LN)], jnp.float32)
                regs = []
                for j in range(LN):
                    e = v * LN + j
                    u = rows_s[e, :] + rows_d[e, :] + attr16[j] * ew_vec
                    m = jnp.maximum(u, 0.2 * u)
                    regs.append(att_vec * m)
                alpha16 = _butterfly(regs, lane, shufxor)
                e16 = jnp.where(valid, jnp.exp(alpha16), 0.0)
                exslab[i, pl.ds(v * LN, LN)] = e16
                plsc.addupdate_scatter(den_v, [d16], e16)

        slots = [(rows_s0, rows_d0, sem_s0, sem_d0),
                 (rows_s1, rows_d1, sem_s1, sem_d1),
                 (rows_s2, rows_d2, sem_s2, sem_d2),
                 (rows_s3, rows_d3, sem_s3, sem_d3)]
        for b, (rs_b, rd_b, ss_b, sd_b) in enumerate(slots):
            start_gathers(slabrow(b), rs_b, rd_b, ss_b, sd_b)

        def it(k, carry):
            for b, (rs_b, rd_b, ss_b, sd_b) in enumerate(slots):
                i = 4 * k + b
                wait_gather(rs_b, ss_b)
                wait_gather(rd_b, sd_b)
                compute_row(i, rs_b, rd_b)

                @pl.when(k < NIT4 - 1)
                def _():
                    start_gathers(slabrow(i + 4), rs_b, rd_b, ss_b, sd_b)
            return carry
        lax.fori_loop(0, NIT4, it, 0)

        pltpu.sync_copy(exslab.at[pl.ds(0, ROWS_BASE)],
                        ex_out.at[pl.ds(r0, ROWS_BASE)])

        @pl.when(cnt > ROWS_BASE)
        def _():
            pltpu.sync_copy(exslab.at[ROWS_BASE],
                            ex_out.at[r0 + ROWS_BASE])

        pltpu.sync_copy(den_v, den_sh.at[sid])
        plsc.subcore_barrier()
        o2 = sid * SLICE
        pltpu.sync_copy(den_sh.at[0, pl.ds(o2, SLICE)], mrg_a)
        for t in range(1, NS):
            pltpu.sync_copy(den_sh.at[t, pl.ds(o2, SLICE)], mrg_b)

            def addv(i, carry):
                for q in range(4):
                    o3 = i * 4 * LN + q * LN
                    mrg_a[pl.ds(o3, LN)] = (
                        mrg_a[pl.ds(o3, LN)] + mrg_b[pl.ds(o3, LN)])
                return carry
            lax.fori_loop(0, SLICE // (4 * LN), addv, 0)
        pltpu.sync_copy(mrg_a, den_out.at[cid, pl.ds(o2, SLICE)])

    return pl.kernel(
        body,
        out_type=[jax.ShapeDtypeStruct((ROWS_TOTAL, EB), jnp.float32),
                  jax.ShapeDtypeStruct((NC, NPAD), jnp.float32)],
        mesh=_mesh(),
        scratch_types=[
            pltpu.VMEM((RPT, 3, EB), jnp.int32),
            pltpu.VMEM((RPT + 1, EB), jnp.float32),
            pltpu.VMEM((EB, WID), jnp.float32),
            pltpu.VMEM((EB, WID), jnp.float32),
            pltpu.VMEM((EB, WID), jnp.float32),
            pltpu.VMEM((EB, WID), jnp.float32),
            pltpu.VMEM((EB, WID), jnp.float32),
            pltpu.VMEM((EB, WID), jnp.float32),
            pltpu.VMEM((EB, WID), jnp.float32),
            pltpu.VMEM((EB, WID), jnp.float32),
            pltpu.VMEM((LN,), jnp.float32),
            pltpu.VMEM((LN,), jnp.float32),
            pltpu.VMEM((NPAD,), jnp.float32),
            pltpu.VMEM((SLICE,), jnp.float32),
            pltpu.VMEM((SLICE,), jnp.float32),
            pltpu.VMEM_SHARED((NS, NPAD), jnp.float32),
        ] + [pltpu.SemaphoreType.DMA] * 8,
        **_SC_PARAMS)


def _edge_pass_b():
    def body(xl_hbm, ep_hbm, ex_hbm, den_hbm, zeros_hbm,
             out_p,
             eslab, exslab, rows_s0, rows_s1, rows_s2, rows_s3,
             scaled0, scaled1, scaled2, scaled3,
             den_v, den_v2, out_sh,
             sem_s0, sem_s1, sem_s2, sem_s3,
             sem_o0, sem_o1, sem_o2, sem_o3):
        cid, sid, r0, cnt = _wid_rows()
        rs = jnp.minimum(r0, ROWS_TOTAL - RPT)
        off = r0 - rs
        pltpu.sync_copy(ep_hbm.at[pl.ds(rs, RPT)], eslab)
        pltpu.sync_copy(ex_hbm.at[pl.ds(rs, RPT)], exslab)
        pltpu.sync_copy(den_hbm.at[0], den_v)
        pltpu.sync_copy(den_hbm.at[1], den_v2)

        def addv(i, carry):
            for q in range(4):
                o3 = i * 4 * LN + q * LN
                den_v[pl.ds(o3, LN)] = (
                    den_v[pl.ds(o3, LN)] + den_v2[pl.ds(o3, LN)])
            return carry
        lax.fori_loop(0, NPAD // (4 * LN), addv, 0)

        o2 = sid * SLICE
        pltpu.sync_copy(zeros_hbm.at[pl.ds(o2, SLICE), :],
                        out_sh.at[pl.ds(o2, SLICE), :])
        plsc.subcore_barrier()

        def slabrow(i):
            return jnp.minimum(off + i, RPT - 1)

        def start_gather(j, rs_buf, ss):
            pltpu.async_copy(xl_hbm.at[eslab.at[j, 0]], rs_buf, ss)

        def wait_gather(buf, sem):
            pltpu.make_async_copy(xl_hbm.at[pl.ds(0, EB)], buf, sem).wait()

        def wait_scatter(buf, j, sem):
            pltpu.make_async_copy(buf, out_sh.at[eslab.at[j, 1]], sem).wait()

        def compute_scaled(i, rows_s, scaled):
            row = slabrow(i)
            valid = i < cnt
            for v in range(EB // LN):
                d16 = eslab[row, 1, pl.ds(v * LN, LN)]
                e16 = exslab[row, pl.ds(v * LN, LN)]
                den16 = plsc.load_gather(den_v, [d16])
                a16 = jnp.where(valid, e16 / (den16 + 1e-16), 0.0)
                for j in range(LN):
                    e = v * LN + j
                    scaled[e, :] = a16[j] * rows_s[e, :]

        slots = [(rows_s0, scaled0, sem_s0, sem_o0),
                 (rows_s1, scaled1, sem_s1, sem_o1),
                 (rows_s2, scaled2, sem_s2, sem_o2),
                 (rows_s3, scaled3, sem_s3, sem_o3)]
        for b, (rs_b, sc_b, ss_b, so_b) in enumerate(slots):
            start_gather(slabrow(b), rs_b, ss_b)

        def it(k, carry):
            for b, (rs_b, sc_b, ss_b, so_b) in enumerate(slots):
                i = 4 * k + b
                wait_gather(rs_b, ss_b)

                @pl.when(k > 0)
                def _():
                    wait_scatter(sc_b, slabrow(i - 4), so_b)
                compute_scaled(i, rs_b, sc_b)
                pltpu.async_copy(sc_b, out_sh.at[eslab.at[slabrow(i), 1]],
                                 so_b, add=True)

                @pl.when(k < NIT4 - 1)
                def _():
                    start_gather(slabrow(i + 4), rs_b, ss_b)
            return carry
        lax.fori_loop(0, NIT4, it, 0)
        for b, (rs_b, sc_b, ss_b, so_b) in enumerate(slots):
            wait_scatter(sc_b, slabrow(4 * NIT4 - 4 + b), so_b)
        plsc.subcore_barrier()
        pltpu.sync_copy(out_sh.at[pl.ds(o2, SLICE), :],
                        out_p.at[cid, pl.ds(o2, SLICE), :])

    return pl.kernel(
        body,
        out_type=[jax.ShapeDtypeStruct((NC, NPAD, WID), jnp.float32)],
        mesh=_mesh(),
        scratch_types=[
            pltpu.VMEM((RPT, 3, EB), jnp.int32),
            pltpu.VMEM((RPT, EB), jnp.float32),
        ] + [pltpu.VMEM((EB, WID), jnp.float32)] * 8 + [
            pltpu.VMEM((NPAD,), jnp.float32),
            pltpu.VMEM((NPAD,), jnp.float32),
            pltpu.VMEM_SHARED((NPAD, WID), jnp.float32),
        ] + [pltpu.SemaphoreType.DMA] * 8,
        **_SC_PARAMS)


def _proj_kernel(x_ref, wl_ref, bl_ref, wr_ref, br_ref, xl_ref, xr_ref):
    xv = x_ref[...]
    xl_ref[...] = jnp.dot(xv, wl_ref[...],
                          preferred_element_type=jnp.float32) + bl_ref[...]
    xr_ref[...] = jnp.dot(xv, wr_ref[...],
                          preferred_element_type=jnp.float32) + br_ref[...]


def _proj(x, wl, bl, wr, br):
    n, d = x.shape
    k = wl.shape[1]
    blk = 1024
    return pl.pallas_call(
        _proj_kernel,
        grid=(n // blk,),
        in_specs=[
            pl.BlockSpec((blk, d), lambda i: (i, 0)),
            pl.BlockSpec((d, k), lambda i: (0, 0)),
            pl.BlockSpec((1, k), lambda i: (0, 0)),
            pl.BlockSpec((d, k), lambda i: (0, 0)),
            pl.BlockSpec((1, k), lambda i: (0, 0)),
        ],
        out_specs=[pl.BlockSpec((blk, k), lambda i: (i, 0)),
                   pl.BlockSpec((blk, k), lambda i: (i, 0))],
        out_shape=[jax.ShapeDtypeStruct((n, k), jnp.float32)] * 2,
    )(x, wl, bl, wr, br)


def _merge_proj_kernel(hin, p_ref, b1_ref, wl_ref, bl_ref, wr_ref, br_ref,
                       xl_ref, xr_ref):
    p = p_ref[0][:, :hin] + p_ref[1][:, :hin]
    h = jnp.maximum(p + b1_ref[...], 0.0)
    xl_ref[...] = jnp.dot(h, wl_ref[...],
                          preferred_element_type=jnp.float32) + bl_ref[...]
    xr_ref[...] = jnp.dot(h, wr_ref[...],
                          preferred_element_type=jnp.float32) + br_ref[...]


def _merge_proj(p, b1, wl, bl, wr, br):
    _, n, wid = p.shape
    hin = wl.shape[0]
    k = wl.shape[1]
    blk = 1024
    return pl.pallas_call(
        functools.partial(_merge_proj_kernel, hin),
        grid=(n // blk,),
        in_specs=[
            pl.BlockSpec((NC, blk, wid), lambda i: (0, i, 0)),
            pl.BlockSpec((1, hin), lambda i: (0, 0)),
            pl.BlockSpec((hin, k), lambda i: (0, 0)),
            pl.BlockSpec((1, k), lambda i: (0, 0)),
            pl.BlockSpec((hin, k), lambda i: (0, 0)),
            pl.BlockSpec((1, k), lambda i: (0, 0)),
        ],
        out_specs=[pl.BlockSpec((blk, k), lambda i: (i, 0)),
                   pl.BlockSpec((blk, k), lambda i: (i, 0))],
        out_shape=[jax.ShapeDtypeStruct((n, k), jnp.float32)] * 2,
    )(p, b1, wl, bl, wr, br)


def _pool_kernel(p_ref, b2_ref, batch_ref, w3t_ref, b3_ref, y_ref,
                 s_acc, c_acc):
    k = pl.program_id(0)

    @pl.when(k == 0)
    def _():
        s_acc[...] = jnp.zeros_like(s_acc)
        c_acc[...] = jnp.zeros_like(c_acc)

    h = jnp.maximum(p_ref[0] + p_ref[1] + b2_ref[...], 0.0)
    b = batch_ref[0, 0, :]
    gi = lax.broadcasted_iota(jnp.int32, (G, PBLK), 0)
    oh = (gi == b[None, :]).astype(jnp.float32)
    s_acc[...] += jnp.dot(oh, h, preferred_element_type=jnp.float32)
    c_acc[...] += jnp.sum(oh, axis=1, keepdims=True)

    @pl.when(k == NBLK - 1)
    def _():
        g = s_acc[...] / jnp.maximum(c_acc[...], 1.0)
        y_ref[...] = jnp.dot(g, w3t_ref[...],
                             preferred_element_type=jnp.float32) + b3_ref[...]


def _pool(p, b2, batch3, w3t, b3):
    hin = p.shape[2]
    return pl.pallas_call(
        _pool_kernel,
        grid=(NBLK,),
        in_specs=[
            pl.BlockSpec((NC, PBLK, hin), lambda i: (0, i, 0)),
            pl.BlockSpec((1, hin), lambda i: (0, 0)),
            pl.BlockSpec((1, 1, PBLK), lambda i: (i, 0, 0)),
            pl.BlockSpec((hin, 1), lambda i: (0, 0)),
            pl.BlockSpec((1, 1), lambda i: (0, 0)),
        ],
        out_specs=pl.BlockSpec((G, 1), lambda i: (0, 0)),
        out_shape=jax.ShapeDtypeStruct((G, 1), jnp.float32),
        scratch_shapes=[pltpu.VMEM((G, hin), jnp.float32),
                        pltpu.VMEM((G, 1), jnp.float32)],
    )(p, b2, batch3, w3t, b3)


def kernel(x, edge_index, edge_attr, batch, lin_l_w1, lin_l_b1, lin_r_w1,
           lin_r_b1, lin_edge_w1, att1, bias1, lin_l_w2, lin_l_b2, lin_r_w2,
           lin_r_b2, lin_edge_w2, att2, bias2, w3, b3):
    n, _ = x.shape
    h1 = lin_l_w1.shape[0]
    h2 = lin_l_w2.shape[0]

    src2d = edge_index[0].reshape(ROWS_TOTAL, EB)
    dst2d = edge_index[1].reshape(ROWS_TOTAL, EB)
    attr2d = lax.bitcast_convert_type(
        edge_attr.reshape(ROWS_TOTAL, EB), jnp.int32)
    epack = jnp.stack([src2d, dst2d, attr2d], axis=1)
    x_pad = jnp.pad(x, ((0, NPAD - n), (0, 0)))
    zeros_nw = jnp.zeros((NPAD, WID), jnp.float32)

    wl1 = jnp.pad(lin_l_w1.T, ((0, 0), (0, WID - h1)))
    wr1 = jnp.pad(lin_r_w1.T, ((0, 0), (0, WID - h1)))
    bl1 = jnp.pad(lin_l_b1, (0, WID - h1))[None]
    br1 = jnp.pad(lin_r_b1, (0, WID - h1))[None]
    xl1, xr1 = _proj(x_pad, wl1, bl1, wr1, br1)
    ewb1 = jnp.pad(lin_edge_w1[:, 0], (0, WID - h1))
    attb1 = jnp.pad(att1, (0, WID - h1))
    ex1, den1 = _edge_pass_a()(xl1, xr1, epack, ewb1, attb1)
    outp1, = _edge_pass_b()(xl1, epack, ex1, den1, zeros_nw)

    xl2, xr2 = _merge_proj(outp1, bias1[None], lin_l_w2.T, lin_l_b2[None],
                           lin_r_w2.T, lin_r_b2[None])
    ewb2 = jnp.pad(lin_edge_w2[:, 0], (0, WID - h2))
    attb2 = jnp.pad(att2, (0, WID - h2))
    ex2, den2 = _edge_pass_a()(xl2, xr2, epack, ewb2, attb2)
    outp2, = _edge_pass_b()(xl2, epack, ex2, den2, zeros_nw)

    y = _pool(outp2[:, :n, :], bias2[None], batch.reshape(NBLK, 1, PBLK),
              w3.T, b3[None])
    return y


# speedup vs baseline: 2.4205x; 1.2349x over previous
import functools

import jax
import jax.numpy as jnp
from jax import lax
from jax.experimental import pallas as pl
from jax.experimental.pallas import tpu as pltpu
from jax.experimental.pallas import tpu_sc as plsc

NC = 2
NS = 16
NW = NC * NS
LN = 16
NPAD = 10240
SLICE = NPAD // NS
EB = 128
ROWS_TOTAL = 2500
ROWS_BASE = ROWS_TOTAL // NW
ROWS_EXTRA = ROWS_TOTAL - ROWS_BASE * NW
RPT = ROWS_BASE + 1
NIT2 = (RPT + 1) // 2
NIT4 = (RPT + 1) // 4
WID = 16
G = 64
PBLK = 1000
NBLK = 10

_SC_PARAMS = dict(
    compiler_params=pltpu.CompilerParams(
        needs_layout_passes=False, use_tc_tiling_on_sc=False))


def _mesh():
    return plsc.VectorSubcoreMesh(
        core_axis_name="c", subcore_axis_name="s",
        num_cores=NC, num_subcores=NS)


def _wid_rows():
    cid = lax.axis_index("c")
    sid = lax.axis_index("s")
    wid = sid * NC + cid
    r0 = wid * ROWS_BASE + jnp.minimum(wid, ROWS_EXTRA)
    cnt = ROWS_BASE + jnp.where(wid < ROWS_EXTRA, 1, 0)
    return cid, sid, r0, cnt


def _zero_1d(ref, n):
    def z(i, carry):
        for q in range(4):
            ref[pl.ds(i * 4 * LN + q * LN, LN)] = jnp.zeros((LN,),
                                                            jnp.float32)
        return carry
    lax.fori_loop(0, n // (4 * LN), z, 0)


def _make_shufxor():
    keys = {sh: jnp.arange(LN, dtype=jnp.int32) ^ sh for sh in (1, 2, 4, 8)}

    def shufxor(x, sh):
        _, out = plsc.sort_key_val(keys[sh], x)
        return out
    return shufxor


def _butterfly(regs, lane, shufxor):
    sh = 1
    while len(regs) > 1:
        msk = (lane & sh) == 0
        nxt = []
        for i2 in range(0, len(regs), 2):
            a, b = regs[i2], regs[i2 + 1]
            c = (jnp.where(msk, a, b)
                 + shufxor(jnp.where(msk, b, a), sh))
            nxt.append(c)
        regs = nxt
        sh *= 2
    return regs[0]


def _edge_pass_a():
    def body(xl_hbm, xr_hbm, ep_hbm, ewb_hbm, attb_hbm,
             ex_out, den_out,
             eslab, exslab, rows_s0, rows_d0, rows_s1, rows_d1, ew_sv,
             att_sv, den_v, mrg_a, mrg_b, den_sh,
             sem_s0, sem_d0, sem_s1, sem_d1):
        cid, sid, r0, cnt = _wid_rows()
        rs = jnp.minimum(r0, ROWS_TOTAL - RPT)
        off = r0 - rs
        pltpu.sync_copy(ep_hbm.at[pl.ds(rs, RPT)], eslab)
        pltpu.sync_copy(ewb_hbm, ew_sv)
        pltpu.sync_copy(attb_hbm, att_sv)
        ew_vec = ew_sv[pl.ds(0, LN)]
        att_vec = att_sv[pl.ds(0, LN)]
        _zero_1d(den_v, NPAD)
        lane = lax.iota(jnp.int32, LN)
        shufxor = _make_shufxor()

        def slabrow(i):
            return jnp.minimum(off + i, RPT - 1)

        def start_gathers(j, rs_buf, rd_buf, ss, sd):
            pltpu.async_copy(xl_hbm.at[eslab.at[j, 0]], rs_buf, ss)
            pltpu.async_copy(xr_hbm.at[eslab.at[j, 1]], rd_buf, sd)

        def wait_gather(buf, sem):
            pltpu.make_async_copy(xl_hbm.at[pl.ds(0, EB)], buf, sem).wait()

        def compute_row(i, rows_s, rows_d):
            row = slabrow(i)
            valid = i < cnt
            for v in range(EB // LN):
                d16 = eslab[row, 1, pl.ds(v * LN, LN)]
                attr16 = plsc.bitcast(
                    eslab[row, 2, pl.ds(v * LN, LN)], jnp.float32)
                regs = []
                for j in range(LN):
                    e = v * LN + j
                    u = rows_s[e, :] + rows_d[e, :] + attr16[j] * ew_vec
                    m = jnp.maximum(u, 0.2 * u)
                    regs.append(att_vec * m)
                alpha16 = _butterfly(regs, lane, shufxor)
                e16 = jnp.where(valid, jnp.exp(alpha16), 0.0)
                exslab[i, pl.ds(v * LN, LN)] = e16
                plsc.addupdate_scatter(den_v, [d16], e16)

        slots = [(rows_s0, rows_d0, sem_s0, sem_d0),
                 (rows_s1, rows_d1, sem_s1, sem_d1)]
        for b, (rs_b, rd_b, ss_b, sd_b) in enumerate(slots):
            start_gathers(slabrow(b), rs_b, rd_b, ss_b, sd_b)

        def it(k, carry):
            for b, (rs_b, rd_b, ss_b, sd_b) in enumerate(slots):
                i = 2 * k + b
                wait_gather(rs_b, ss_b)
                wait_gather(rd_b, sd_b)
                compute_row(i, rs_b, rd_b)

                @pl.when(k < NIT2 - 1)
                def _():
                    start_gathers(slabrow(i + 2), rs_b, rd_b, ss_b, sd_b)
            return carry
        lax.fori_loop(0, NIT2, it, 0)

        pltpu.sync_copy(exslab.at[pl.ds(0, ROWS_BASE)],
                        ex_out.at[pl.ds(r0, ROWS_BASE)])

        @pl.when(cnt > ROWS_BASE)
        def _():
            pltpu.sync_copy(exslab.at[ROWS_BASE],
                            ex_out.at[r0 + ROWS_BASE])

        pltpu.sync_copy(den_v, den_sh.at[sid])
        plsc.subcore_barrier()
        o2 = sid * SLICE
        pltpu.sync_copy(den_sh.at[0, pl.ds(o2, SLICE)], mrg_a)
        for t in range(1, NS):
            pltpu.sync_copy(den_sh.at[t, pl.ds(o2, SLICE)], mrg_b)

            def addv(i, carry):
                for q in range(4):
                    o3 = i * 4 * LN + q * LN
                    mrg_a[pl.ds(o3, LN)] = (
                        mrg_a[pl.ds(o3, LN)] + mrg_b[pl.ds(o3, LN)])
                return carry
            lax.fori_loop(0, SLICE // (4 * LN), addv, 0)
        pltpu.sync_copy(mrg_a, den_out.at[cid, pl.ds(o2, SLICE)])

    return pl.kernel(
        body,
        out_type=[jax.ShapeDtypeStruct((ROWS_TOTAL, EB), jnp.float32),
                  jax.ShapeDtypeStruct((NC, NPAD), jnp.float32)],
        mesh=_mesh(),
        scratch_types=[
            pltpu.VMEM((RPT, 3, EB), jnp.int32),
            pltpu.VMEM((RPT + 1, EB), jnp.float32),
            pltpu.VMEM((EB, WID), jnp.float32),
            pltpu.VMEM((EB, WID), jnp.float32),
            pltpu.VMEM((EB, WID), jnp.float32),
            pltpu.VMEM((EB, WID), jnp.float32),
            pltpu.VMEM((LN,), jnp.float32),
            pltpu.VMEM((LN,), jnp.float32),
            pltpu.VMEM((NPAD,), jnp.float32),
            pltpu.VMEM((SLICE,), jnp.float32),
            pltpu.VMEM((SLICE,), jnp.float32),
            pltpu.VMEM_SHARED((NS, NPAD), jnp.float32),
        ] + [pltpu.SemaphoreType.DMA] * 4,
        **_SC_PARAMS)


def _edge_pass_b():
    def body(xl_hbm, ep_hbm, ex_hbm, den_hbm, zeros_hbm,
             out_p,
             eslab, exslab, rows_s0, rows_s1, rows_s2, rows_s3,
             scaled0, scaled1, scaled2, scaled3,
             den_v, den_v2, out_sh,
             sem_s0, sem_s1, sem_s2, sem_s3,
             sem_o0, sem_o1, sem_o2, sem_o3):
        cid, sid, r0, cnt = _wid_rows()
        rs = jnp.minimum(r0, ROWS_TOTAL - RPT)
        off = r0 - rs
        pltpu.sync_copy(ep_hbm.at[pl.ds(rs, RPT)], eslab)
        pltpu.sync_copy(ex_hbm.at[pl.ds(rs, RPT)], exslab)
        pltpu.sync_copy(den_hbm.at[0], den_v)
        pltpu.sync_copy(den_hbm.at[1], den_v2)

        def addv(i, carry):
            for q in range(4):
                o3 = i * 4 * LN + q * LN
                den_v[pl.ds(o3, LN)] = (
                    den_v[pl.ds(o3, LN)] + den_v2[pl.ds(o3, LN)])
            return carry
        lax.fori_loop(0, NPAD // (4 * LN), addv, 0)

        o2 = sid * SLICE
        pltpu.sync_copy(zeros_hbm.at[pl.ds(o2, SLICE), :],
                        out_sh.at[pl.ds(o2, SLICE), :])
        plsc.subcore_barrier()

        def slabrow(i):
            return jnp.minimum(off + i, RPT - 1)

        def start_gather(j, rs_buf, ss):
            pltpu.async_copy(xl_hbm.at[eslab.at[j, 0]], rs_buf, ss)

        def wait_gather(buf, sem):
            pltpu.make_async_copy(xl_hbm.at[pl.ds(0, EB)], buf, sem).wait()

        def wait_scatter(buf, j, sem):
            pltpu.make_async_copy(buf, out_sh.at[eslab.at[j, 1]], sem).wait()

        def compute_scaled(i, rows_s, scaled):
            row = slabrow(i)
            valid = i < cnt
            for v in range(EB // LN):
                d16 = eslab[row, 1, pl.ds(v * LN, LN)]
                e16 = exslab[row, pl.ds(v * LN, LN)]
                den16 = plsc.load_gather(den_v, [d16])
                a16 = jnp.where(valid, e16 / (den16 + 1e-16), 0.0)
                for j in range(LN):
                    e = v * LN + j
                    scaled[e, :] = a16[j] * rows_s[e, :]

        slots = [(rows_s0, scaled0, sem_s0, sem_o0),
                 (rows_s1, scaled1, sem_s1, sem_o1),
                 (rows_s2, scaled2, sem_s2, sem_o2),
                 (rows_s3, scaled3, sem_s3, sem_o3)]
        for b, (rs_b, sc_b, ss_b, so_b) in enumerate(slots):
            start_gather(slabrow(b), rs_b, ss_b)

        def it(k, carry):
            for b, (rs_b, sc_b, ss_b, so_b) in enumerate(slots):
                i = 4 * k + b
                wait_gather(rs_b, ss_b)

                @pl.when(k > 0)
                def _():
                    wait_scatter(sc_b, slabrow(i - 4), so_b)
                compute_scaled(i, rs_b, sc_b)
                pltpu.async_copy(sc_b, out_sh.at[eslab.at[slabrow(i), 1]],
                                 so_b, add=True)

                @pl.when(k < NIT4 - 1)
                def _():
                    start_gather(slabrow(i + 4), rs_b, ss_b)
            return carry
        lax.fori_loop(0, NIT4, it, 0)
        for b, (rs_b, sc_b, ss_b, so_b) in enumerate(slots):
            wait_scatter(sc_b, slabrow(4 * NIT4 - 4 + b), so_b)
        plsc.subcore_barrier()
        pltpu.sync_copy(out_sh.at[pl.ds(o2, SLICE), :],
                        out_p.at[cid, pl.ds(o2, SLICE), :])

    return pl.kernel(
        body,
        out_type=[jax.ShapeDtypeStruct((NC, NPAD, WID), jnp.float32)],
        mesh=_mesh(),
        scratch_types=[
            pltpu.VMEM((RPT, 3, EB), jnp.int32),
            pltpu.VMEM((RPT, EB), jnp.float32),
        ] + [pltpu.VMEM((EB, WID), jnp.float32)] * 8 + [
            pltpu.VMEM((NPAD,), jnp.float32),
            pltpu.VMEM((NPAD,), jnp.float32),
            pltpu.VMEM_SHARED((NPAD, WID), jnp.float32),
        ] + [pltpu.SemaphoreType.DMA] * 8,
        **_SC_PARAMS)


def _proj_kernel(x_ref, wl_ref, bl_ref, wr_ref, br_ref, xl_ref, xr_ref):
    xv = x_ref[...]
    xl_ref[...] = jnp.dot(xv, wl_ref[...],
                          preferred_element_type=jnp.float32) + bl_ref[...]
    xr_ref[...] = jnp.dot(xv, wr_ref[...],
                          preferred_element_type=jnp.float32) + br_ref[...]


def _proj(x, wl, bl, wr, br):
    n, d = x.shape
    k = wl.shape[1]
    blk = 1024
    return pl.pallas_call(
        _proj_kernel,
        grid=(n // blk,),
        in_specs=[
            pl.BlockSpec((blk, d), lambda i: (i, 0)),
            pl.BlockSpec((d, k), lambda i: (0, 0)),
            pl.BlockSpec((1, k), lambda i: (0, 0)),
            pl.BlockSpec((d, k), lambda i: (0, 0)),
            pl.BlockSpec((1, k), lambda i: (0, 0)),
        ],
        out_specs=[pl.BlockSpec((blk, k), lambda i: (i, 0)),
                   pl.BlockSpec((blk, k), lambda i: (i, 0))],
        out_shape=[jax.ShapeDtypeStruct((n, k), jnp.float32)] * 2,
    )(x, wl, bl, wr, br)


def _merge_proj_kernel(hin, p_ref, b1_ref, wl_ref, bl_ref, wr_ref, br_ref,
                       xl_ref, xr_ref):
    p = p_ref[0][:, :hin] + p_ref[1][:, :hin]
    h = jnp.maximum(p + b1_ref[...], 0.0)
    xl_ref[...] = jnp.dot(h, wl_ref[...],
                          preferred_element_type=jnp.float32) + bl_ref[...]
    xr_ref[...] = jnp.dot(h, wr_ref[...],
                          preferred_element_type=jnp.float32) + br_ref[...]


def _merge_proj(p, b1, wl, bl, wr, br):
    _, n, wid = p.shape
    hin = wl.shape[0]
    k = wl.shape[1]
    blk = 1024
    return pl.pallas_call(
        functools.partial(_merge_proj_kernel, hin),
        grid=(n // blk,),
        in_specs=[
            pl.BlockSpec((NC, blk, wid), lambda i: (0, i, 0)),
            pl.BlockSpec((1, hin), lambda i: (0, 0)),
            pl.BlockSpec((hin, k), lambda i: (0, 0)),
            pl.BlockSpec((1, k), lambda i: (0, 0)),
            pl.BlockSpec((hin, k), lambda i: (0, 0)),
            pl.BlockSpec((1, k), lambda i: (0, 0)),
        ],
        out_specs=[pl.BlockSpec((blk, k), lambda i: (i, 0)),
                   pl.BlockSpec((blk, k), lambda i: (i, 0))],
        out_shape=[jax.ShapeDtypeStruct((n, k), jnp.float32)] * 2,
    )(p, b1, wl, bl, wr, br)


def _pool_kernel(p_ref, b2_ref, batch_ref, w3t_ref, b3_ref, y_ref,
                 s_acc, c_acc):
    k = pl.program_id(0)

    @pl.when(k == 0)
    def _():
        s_acc[...] = jnp.zeros_like(s_acc)
        c_acc[...] = jnp.zeros_like(c_acc)

    h = jnp.maximum(p_ref[0] + p_ref[1] + b2_ref[...], 0.0)
    b = batch_ref[0, 0, :]
    gi = lax.broadcasted_iota(jnp.int32, (G, PBLK), 0)
    oh = (gi == b[None, :]).astype(jnp.float32)
    s_acc[...] += jnp.dot(oh, h, preferred_element_type=jnp.float32)
    c_acc[...] += jnp.sum(oh, axis=1, keepdims=True)

    @pl.when(k == NBLK - 1)
    def _():
        g = s_acc[...] / jnp.maximum(c_acc[...], 1.0)
        y_ref[...] = jnp.dot(g, w3t_ref[...],
                             preferred_element_type=jnp.float32) + b3_ref[...]


def _pool(p, b2, batch3, w3t, b3):
    hin = p.shape[2]
    return pl.pallas_call(
        _pool_kernel,
        grid=(NBLK,),
        in_specs=[
            pl.BlockSpec((NC, PBLK, hin), lambda i: (0, i, 0)),
            pl.BlockSpec((1, hin), lambda i: (0, 0)),
            pl.BlockSpec((1, 1, PBLK), lambda i: (i, 0, 0)),
            pl.BlockSpec((hin, 1), lambda i: (0, 0)),
            pl.BlockSpec((1, 1), lambda i: (0, 0)),
        ],
        out_specs=pl.BlockSpec((G, 1), lambda i: (0, 0)),
        out_shape=jax.ShapeDtypeStruct((G, 1), jnp.float32),
        scratch_shapes=[pltpu.VMEM((G, hin), jnp.float32),
                        pltpu.VMEM((G, 1), jnp.float32)],
    )(p, b2, batch3, w3t, b3)


def kernel(x, edge_index, edge_attr, batch, lin_l_w1, lin_l_b1, lin_r_w1,
           lin_r_b1, lin_edge_w1, att1, bias1, lin_l_w2, lin_l_b2, lin_r_w2,
           lin_r_b2, lin_edge_w2, att2, bias2, w3, b3):
    n, _ = x.shape
    h1 = lin_l_w1.shape[0]
    h2 = lin_l_w2.shape[0]

    src2d = edge_index[0].reshape(ROWS_TOTAL, EB)
    dst2d = edge_index[1].reshape(ROWS_TOTAL, EB)
    attr2d = lax.bitcast_convert_type(
        edge_attr.reshape(ROWS_TOTAL, EB), jnp.int32)
    epack = jnp.stack([src2d, dst2d, attr2d], axis=1)
    x_pad = jnp.pad(x, ((0, NPAD - n), (0, 0)))
    zeros_nw = jnp.zeros((NPAD, WID), jnp.float32)

    wl1 = jnp.pad(lin_l_w1.T, ((0, 0), (0, WID - h1)))
    wr1 = jnp.pad(lin_r_w1.T, ((0, 0), (0, WID - h1)))
    bl1 = jnp.pad(lin_l_b1, (0, WID - h1))[None]
    br1 = jnp.pad(lin_r_b1, (0, WID - h1))[None]
    xl1, xr1 = _proj(x_pad, wl1, bl1, wr1, br1)
    ewb1 = jnp.pad(lin_edge_w1[:, 0], (0, WID - h1))
    attb1 = jnp.pad(att1, (0, WID - h1))
    ex1, den1 = _edge_pass_a()(xl1, xr1, epack, ewb1, attb1)
    outp1, = _edge_pass_b()(xl1, epack, ex1, den1, zeros_nw)

    xl2, xr2 = _merge_proj(outp1, bias1[None], lin_l_w2.T, lin_l_b2[None],
                           lin_r_w2.T, lin_r_b2[None])
    ewb2 = jnp.pad(lin_edge_w2[:, 0], (0, WID - h2))
    attb2 = jnp.pad(att2, (0, WID - h2))
    ex2, den2 = _edge_pass_a()(xl2, xr2, epack, ewb2, attb2)
    outp2, = _edge_pass_b()(xl2, epack, ex2, den2, zeros_nw)

    y = _pool(outp2[:, :n, :], bias2[None], batch.reshape(NBLK, 1, PBLK),
              w3.T, b3[None])
    return y


# speedup vs baseline: 2.4710x; 1.0209x over previous
import functools

import jax
import jax.numpy as jnp
from jax import lax
from jax.experimental import pallas as pl
from jax.experimental.pallas import tpu as pltpu
from jax.experimental.pallas import tpu_sc as plsc

NC = 2
NS = 16
NW = NC * NS
LN = 16
NPAD = 10240
SLICE = NPAD // NS
EB = 128
ROWS_TOTAL = 2500
ROWS_BASE = ROWS_TOTAL // NW
ROWS_EXTRA = ROWS_TOTAL - ROWS_BASE * NW
RPT = ROWS_BASE + 1
NIT2 = (RPT + 1) // 2
NIT4 = (RPT + 1) // 4
WID = 16
G = 64
PBLK = 1000
NBLK = 10

_SC_PARAMS = dict(
    compiler_params=pltpu.CompilerParams(
        needs_layout_passes=False, use_tc_tiling_on_sc=False))


def _mesh():
    return plsc.VectorSubcoreMesh(
        core_axis_name="c", subcore_axis_name="s",
        num_cores=NC, num_subcores=NS)


def _wid_rows():
    cid = lax.axis_index("c")
    sid = lax.axis_index("s")
    wid = sid * NC + cid
    r0 = wid * ROWS_BASE + jnp.minimum(wid, ROWS_EXTRA)
    cnt = ROWS_BASE + jnp.where(wid < ROWS_EXTRA, 1, 0)
    return cid, sid, r0, cnt


def _zero_1d(ref, n):
    def z(i, carry):
        for q in range(4):
            ref[pl.ds(i * 4 * LN + q * LN, LN)] = jnp.zeros((LN,),
                                                            jnp.float32)
        return carry
    lax.fori_loop(0, n // (4 * LN), z, 0)


def _make_shufxor():
    keys = {sh: jnp.arange(LN, dtype=jnp.int32) ^ sh for sh in (1, 2, 4, 8)}

    def shufxor(x, sh):
        _, out = plsc.sort_key_val(keys[sh], x)
        return out
    return shufxor


def _butterfly(regs, lane, shufxor):
    sh = 1
    while len(regs) > 1:
        msk = (lane & sh) == 0
        nxt = []
        for i2 in range(0, len(regs), 2):
            a, b = regs[i2], regs[i2 + 1]
            c = (jnp.where(msk, a, b)
                 + shufxor(jnp.where(msk, b, a), sh))
            nxt.append(c)
        regs = nxt
        sh *= 2
    return regs[0]


def _edge_pass_a():
    def body(xl_hbm, xr_hbm, ep_hbm, ewb_hbm, attb_hbm,
             ex_out, den_out,
             eslab, exslab, rows_s0, rows_d0, rows_s1, rows_d1, ew_sv,
             att_sv, den_v, mrg_a, mrg2d, den_sh,
             sem_s0, sem_d0, sem_s1, sem_d1):
        cid, sid, r0, cnt = _wid_rows()
        rs = jnp.minimum(r0, ROWS_TOTAL - RPT)
        off = r0 - rs
        pltpu.sync_copy(ep_hbm.at[pl.ds(rs, RPT)], eslab)
        pltpu.sync_copy(ewb_hbm, ew_sv)
        pltpu.sync_copy(attb_hbm, att_sv)
        ew_vec = ew_sv[pl.ds(0, LN)]
        att_vec = att_sv[pl.ds(0, LN)]
        _zero_1d(den_v, NPAD)
        lane = lax.iota(jnp.int32, LN)
        shufxor = _make_shufxor()

        def slabrow(i):
            return jnp.minimum(off + i, RPT - 1)

        def start_gathers(j, rs_buf, rd_buf, ss, sd):
            pltpu.async_copy(xl_hbm.at[eslab.at[j, 0]], rs_buf, ss)
            pltpu.async_copy(xr_hbm.at[eslab.at[j, 1]], rd_buf, sd)

        def wait_gather(buf, sem):
            pltpu.make_async_copy(xl_hbm.at[pl.ds(0, EB)], buf, sem).wait()

        def compute_row(i, rows_s, rows_d):
            row = slabrow(i)
            valid = i < cnt
            for v in range(EB // LN):
                d16 = eslab[row, 1, pl.ds(v * LN, LN)]
                attr16 = plsc.bitcast(
                    eslab[row, 2, pl.ds(v * LN, LN)], jnp.float32)
                regs = []
                for j in range(LN):
                    e = v * LN + j
                    u = rows_s[e, :] + rows_d[e, :] + attr16[j] * ew_vec
                    m = jnp.maximum(u, 0.2 * u)
                    regs.append(att_vec * m)
                alpha16 = _butterfly(regs, lane, shufxor)
                e16 = jnp.where(valid, jnp.exp(alpha16), 0.0)
                exslab[i, pl.ds(v * LN, LN)] = e16
                plsc.addupdate_scatter(den_v, [d16], e16)

        slots = [(rows_s0, rows_d0, sem_s0, sem_d0),
                 (rows_s1, rows_d1, sem_s1, sem_d1)]
        for b, (rs_b, rd_b, ss_b, sd_b) in enumerate(slots):
            start_gathers(slabrow(b), rs_b, rd_b, ss_b, sd_b)

        def it(k, carry):
            for b, (rs_b, rd_b, ss_b, sd_b) in enumerate(slots):
                i = 2 * k + b
                wait_gather(rs_b, ss_b)
                wait_gather(rd_b, sd_b)
                compute_row(i, rs_b, rd_b)

                @pl.when(k < NIT2 - 1)
                def _():
                    start_gathers(slabrow(i + 2), rs_b, rd_b, ss_b, sd_b)
            return carry
        lax.fori_loop(0, NIT2, it, 0)

        pltpu.sync_copy(exslab.at[pl.ds(0, ROWS_BASE)],
                        ex_out.at[pl.ds(r0, ROWS_BASE)])

        @pl.when(cnt > ROWS_BASE)
        def _():
            pltpu.sync_copy(exslab.at[ROWS_BASE],
                            ex_out.at[r0 + ROWS_BASE])

        pltpu.sync_copy(den_v, den_sh.at[sid])
        plsc.subcore_barrier()
        o2 = sid * SLICE
        pltpu.sync_copy(den_sh.at[:, pl.ds(o2, SLICE)], mrg2d)

        def addv(i, carry):
            for q in range(4):
                o3 = i * 4 * LN + q * LN
                acc = mrg2d[0, pl.ds(o3, LN)]
                for t in range(1, NS):
                    acc = acc + mrg2d[t, pl.ds(o3, LN)]
                mrg_a[pl.ds(o3, LN)] = acc
            return carry
        lax.fori_loop(0, SLICE // (4 * LN), addv, 0)
        pltpu.sync_copy(mrg_a, den_out.at[cid, pl.ds(o2, SLICE)])

    return pl.kernel(
        body,
        out_type=[jax.ShapeDtypeStruct((ROWS_TOTAL, EB), jnp.float32),
                  jax.ShapeDtypeStruct((NC, NPAD), jnp.float32)],
        mesh=_mesh(),
        scratch_types=[
            pltpu.VMEM((RPT, 3, EB), jnp.int32),
            pltpu.VMEM((RPT + 1, EB), jnp.float32),
            pltpu.VMEM((EB, WID), jnp.float32),
            pltpu.VMEM((EB, WID), jnp.float32),
            pltpu.VMEM((EB, WID), jnp.float32),
            pltpu.VMEM((EB, WID), jnp.float32),
            pltpu.VMEM((LN,), jnp.float32),
            pltpu.VMEM((LN,), jnp.float32),
            pltpu.VMEM((NPAD,), jnp.float32),
            pltpu.VMEM((SLICE,), jnp.float32),
            pltpu.VMEM((NS, SLICE), jnp.float32),
            pltpu.VMEM_SHARED((NS, NPAD), jnp.float32),
        ] + [pltpu.SemaphoreType.DMA] * 4,
        **_SC_PARAMS)


def _edge_pass_b():
    def body(xl_hbm, ep_hbm, ex_hbm, den_hbm, zeros_hbm,
             out_p,
             eslab, exslab, rows_s0, rows_s1, rows_s2, rows_s3,
             scaled0, scaled1, scaled2, scaled3,
             den_v, den_v2, out_sh,
             sem_s0, sem_s1, sem_s2, sem_s3,
             sem_o0, sem_o1, sem_o2, sem_o3):
        cid, sid, r0, cnt = _wid_rows()
        rs = jnp.minimum(r0, ROWS_TOTAL - RPT)
        off = r0 - rs
        pltpu.sync_copy(ep_hbm.at[pl.ds(rs, RPT)], eslab)
        pltpu.sync_copy(ex_hbm.at[pl.ds(rs, RPT)], exslab)
        pltpu.sync_copy(den_hbm.at[0], den_v)
        pltpu.sync_copy(den_hbm.at[1], den_v2)

        def addv(i, carry):
            for q in range(4):
                o3 = i * 4 * LN + q * LN
                den_v[pl.ds(o3, LN)] = (
                    den_v[pl.ds(o3, LN)] + den_v2[pl.ds(o3, LN)])
            return carry
        lax.fori_loop(0, NPAD // (4 * LN), addv, 0)

        o2 = sid * SLICE
        pltpu.sync_copy(zeros_hbm.at[pl.ds(o2, SLICE), :],
                        out_sh.at[pl.ds(o2, SLICE), :])
        plsc.subcore_barrier()

        def slabrow(i):
            return jnp.minimum(off + i, RPT - 1)

        def start_gather(j, rs_buf, ss):
            pltpu.async_copy(xl_hbm.at[eslab.at[j, 0]], rs_buf, ss)

        def wait_gather(buf, sem):
            pltpu.make_async_copy(xl_hbm.at[pl.ds(0, EB)], buf, sem).wait()

        def wait_scatter(buf, j, sem):
            pltpu.make_async_copy(buf, out_sh.at[eslab.at[j, 1]], sem).wait()

        def compute_scaled(i, rows_s, scaled):
            row = slabrow(i)
            valid = i < cnt
            for v in range(EB // LN):
                d16 = eslab[row, 1, pl.ds(v * LN, LN)]
                e16 = exslab[row, pl.ds(v * LN, LN)]
                den16 = plsc.load_gather(den_v, [d16])
                a16 = jnp.where(valid, e16 / (den16 + 1e-16), 0.0)
                for j in range(LN):
                    e = v * LN + j
                    scaled[e, :] = a16[j] * rows_s[e, :]

        slots = [(rows_s0, scaled0, sem_s0, sem_o0),
                 (rows_s1, scaled1, sem_s1, sem_o1),
                 (rows_s2, scaled2, sem_s2, sem_o2),
                 (rows_s3, scaled3, sem_s3, sem_o3)]
        for b, (rs_b, sc_b, ss_b, so_b) in enumerate(slots):
            start_gather(slabrow(b), rs_b, ss_b)

        def it(k, carry):
            for b, (rs_b, sc_b, ss_b, so_b) in enumerate(slots):
                i = 4 * k + b
                wait_gather(rs_b, ss_b)

                @pl.when(k > 0)
                def _():
                    wait_scatter(sc_b, slabrow(i - 4), so_b)
                compute_scaled(i, rs_b, sc_b)
                pltpu.async_copy(sc_b, out_sh.at[eslab.at[slabrow(i), 1]],
                                 so_b, add=True)

                @pl.when(k < NIT4 - 1)
                def _():
                    start_gather(slabrow(i + 4), rs_b, ss_b)
            return carry
        lax.fori_loop(0, NIT4, it, 0)
        for b, (rs_b, sc_b, ss_b, so_b) in enumerate(slots):
            wait_scatter(sc_b, slabrow(4 * NIT4 - 4 + b), so_b)
        plsc.subcore_barrier()
        pltpu.sync_copy(out_sh.at[pl.ds(o2, SLICE), :],
                        out_p.at[cid, pl.ds(o2, SLICE), :])

    return pl.kernel(
        body,
        out_type=[jax.ShapeDtypeStruct((NC, NPAD, WID), jnp.float32)],
        mesh=_mesh(),
        scratch_types=[
            pltpu.VMEM((RPT, 3, EB), jnp.int32),
            pltpu.VMEM((RPT, EB), jnp.float32),
        ] + [pltpu.VMEM((EB, WID), jnp.float32)] * 8 + [
            pltpu.VMEM((NPAD,), jnp.float32),
            pltpu.VMEM((NPAD,), jnp.float32),
            pltpu.VMEM_SHARED((NPAD, WID), jnp.float32),
        ] + [pltpu.SemaphoreType.DMA] * 8,
        **_SC_PARAMS)


def _proj_kernel(x_ref, wl_ref, bl_ref, wr_ref, br_ref, xl_ref, xr_ref):
    xv = x_ref[...]
    xl_ref[...] = jnp.dot(xv, wl_ref[...],
                          preferred_element_type=jnp.float32) + bl_ref[...]
    xr_ref[...] = jnp.dot(xv, wr_ref[...],
                          preferred_element_type=jnp.float32) + br_ref[...]


def _proj(x, wl, bl, wr, br):
    n, d = x.shape
    k = wl.shape[1]
    blk = 1024
    return pl.pallas_call(
        _proj_kernel,
        grid=(n // blk,),
        in_specs=[
            pl.BlockSpec((blk, d), lambda i: (i, 0)),
            pl.BlockSpec((d, k), lambda i: (0, 0)),
            pl.BlockSpec((1, k), lambda i: (0, 0)),
            pl.BlockSpec((d, k), lambda i: (0, 0)),
            pl.BlockSpec((1, k), lambda i: (0, 0)),
        ],
        out_specs=[pl.BlockSpec((blk, k), lambda i: (i, 0)),
                   pl.BlockSpec((blk, k), lambda i: (i, 0))],
        out_shape=[jax.ShapeDtypeStruct((n, k), jnp.float32)] * 2,
    )(x, wl, bl, wr, br)


def _merge_proj_kernel(hin, p_ref, b1_ref, wl_ref, bl_ref, wr_ref, br_ref,
                       xl_ref, xr_ref):
    p = p_ref[0][:, :hin] + p_ref[1][:, :hin]
    h = jnp.maximum(p + b1_ref[...], 0.0)
    xl_ref[...] = jnp.dot(h, wl_ref[...],
                          preferred_element_type=jnp.float32) + bl_ref[...]
    xr_ref[...] = jnp.dot(h, wr_ref[...],
                          preferred_element_type=jnp.float32) + br_ref[...]


def _merge_proj(p, b1, wl, bl, wr, br):
    _, n, wid = p.shape
    hin = wl.shape[0]
    k = wl.shape[1]
    blk = 1024
    return pl.pallas_call(
        functools.partial(_merge_proj_kernel, hin),
        grid=(n // blk,),
        in_specs=[
            pl.BlockSpec((NC, blk, wid), lambda i: (0, i, 0)),
            pl.BlockSpec((1, hin), lambda i: (0, 0)),
            pl.BlockSpec((hin, k), lambda i: (0, 0)),
            pl.BlockSpec((1, k), lambda i: (0, 0)),
            pl.BlockSpec((hin, k), lambda i: (0, 0)),
            pl.BlockSpec((1, k), lambda i: (0, 0)),
        ],
        out_specs=[pl.BlockSpec((blk, k), lambda i: (i, 0)),
                   pl.BlockSpec((blk, k), lambda i: (i, 0))],
        out_shape=[jax.ShapeDtypeStruct((n, k), jnp.float32)] * 2,
    )(p, b1, wl, bl, wr, br)


def _pool_kernel(p_ref, b2_ref, batch_ref, w3t_ref, b3_ref, y_ref,
                 s_acc, c_acc):
    k = pl.program_id(0)

    @pl.when(k == 0)
    def _():
        s_acc[...] = jnp.zeros_like(s_acc)
        c_acc[...] = jnp.zeros_like(c_acc)

    h = jnp.maximum(p_ref[0] + p_ref[1] + b2_ref[...], 0.0)
    b = batch_ref[0, 0, :]
    gi = lax.broadcasted_iota(jnp.int32, (G, PBLK), 0)
    oh = (gi == b[None, :]).astype(jnp.float32)
    s_acc[...] += jnp.dot(oh, h, preferred_element_type=jnp.float32)
    c_acc[...] += jnp.sum(oh, axis=1, keepdims=True)

    @pl.when(k == NBLK - 1)
    def _():
        g = s_acc[...] / jnp.maximum(c_acc[...], 1.0)
        y_ref[...] = jnp.dot(g, w3t_ref[...],
                             preferred_element_type=jnp.float32) + b3_ref[...]


def _pool(p, b2, batch3, w3t, b3):
    hin = p.shape[2]
    return pl.pallas_call(
        _pool_kernel,
        grid=(NBLK,),
        in_specs=[
            pl.BlockSpec((NC, PBLK, hin), lambda i: (0, i, 0)),
            pl.BlockSpec((1, hin), lambda i: (0, 0)),
            pl.BlockSpec((1, 1, PBLK), lambda i: (i, 0, 0)),
            pl.BlockSpec((hin, 1), lambda i: (0, 0)),
            pl.BlockSpec((1, 1), lambda i: (0, 0)),
        ],
        out_specs=pl.BlockSpec((G, 1), lambda i: (0, 0)),
        out_shape=jax.ShapeDtypeStruct((G, 1), jnp.float32),
        scratch_shapes=[pltpu.VMEM((G, hin), jnp.float32),
                        pltpu.VMEM((G, 1), jnp.float32)],
    )(p, b2, batch3, w3t, b3)


def kernel(x, edge_index, edge_attr, batch, lin_l_w1, lin_l_b1, lin_r_w1,
           lin_r_b1, lin_edge_w1, att1, bias1, lin_l_w2, lin_l_b2, lin_r_w2,
           lin_r_b2, lin_edge_w2, att2, bias2, w3, b3):
    n, _ = x.shape
    h1 = lin_l_w1.shape[0]
    h2 = lin_l_w2.shape[0]

    src2d = edge_index[0].reshape(ROWS_TOTAL, EB)
    dst2d = edge_index[1].reshape(ROWS_TOTAL, EB)
    attr2d = lax.bitcast_convert_type(
        edge_attr.reshape(ROWS_TOTAL, EB), jnp.int32)
    epack = jnp.stack([src2d, dst2d, attr2d], axis=1)
    x_pad = jnp.pad(x, ((0, NPAD - n), (0, 0)))
    zeros_nw = jnp.zeros((NPAD, WID), jnp.float32)

    wl1 = jnp.pad(lin_l_w1.T, ((0, 0), (0, WID - h1)))
    wr1 = jnp.pad(lin_r_w1.T, ((0, 0), (0, WID - h1)))
    bl1 = jnp.pad(lin_l_b1, (0, WID - h1))[None]
    br1 = jnp.pad(lin_r_b1, (0, WID - h1))[None]
    xl1, xr1 = _proj(x_pad, wl1, bl1, wr1, br1)
    ewb1 = jnp.pad(lin_edge_w1[:, 0], (0, WID - h1))
    attb1 = jnp.pad(att1, (0, WID - h1))
    ex1, den1 = _edge_pass_a()(xl1, xr1, epack, ewb1, attb1)
    outp1, = _edge_pass_b()(xl1, epack, ex1, den1, zeros_nw)

    xl2, xr2 = _merge_proj(outp1, bias1[None], lin_l_w2.T, lin_l_b2[None],
                           lin_r_w2.T, lin_r_b2[None])
    ewb2 = jnp.pad(lin_edge_w2[:, 0], (0, WID - h2))
    attb2 = jnp.pad(att2, (0, WID - h2))
    ex2, den2 = _edge_pass_a()(xl2, xr2, epack, ewb2, attb2)
    outp2, = _edge_pass_b()(xl2, epack, ex2, den2, zeros_nw)

    y = _pool(outp2[:, :n, :], bias2[None], batch.reshape(NBLK, 1, PBLK),
              w3.T, b3[None])
    return y


# speedup vs baseline: 2.4984x; 1.0111x over previous
import functools

import jax
import jax.numpy as jnp
from jax import lax
from jax.experimental import pallas as pl
from jax.experimental.pallas import tpu as pltpu
from jax.experimental.pallas import tpu_sc as plsc

NC = 2
NS = 16
NW = NC * NS
LN = 16
NPAD = 10240
SLICE = NPAD // NS
EB = 128
ROWS_TOTAL = 2500
ROWS_BASE = ROWS_TOTAL // NW
ROWS_EXTRA = ROWS_TOTAL - ROWS_BASE * NW
RPT = ROWS_BASE + 1
NIT2 = (RPT + 1) // 2
NIT4 = (RPT + 1) // 4
WID = 16
G = 64
PBLK = 1000
NBLK = 10

_SC_PARAMS = dict(
    compiler_params=pltpu.CompilerParams(
        needs_layout_passes=False, use_tc_tiling_on_sc=False))


def _mesh():
    return plsc.VectorSubcoreMesh(
        core_axis_name="c", subcore_axis_name="s",
        num_cores=NC, num_subcores=NS)


def _wid_rows():
    cid = lax.axis_index("c")
    sid = lax.axis_index("s")
    wid = sid * NC + cid
    r0 = wid * ROWS_BASE + jnp.minimum(wid, ROWS_EXTRA)
    cnt = ROWS_BASE + jnp.where(wid < ROWS_EXTRA, 1, 0)
    return cid, sid, r0, cnt


def _zero_1d(ref, n):
    def z(i, carry):
        for q in range(4):
            ref[pl.ds(i * 4 * LN + q * LN, LN)] = jnp.zeros((LN,),
                                                            jnp.float32)
        return carry
    lax.fori_loop(0, n // (4 * LN), z, 0)


def _make_shufxor():
    keys = {sh: jnp.arange(LN, dtype=jnp.int32) ^ sh for sh in (1, 2, 4, 8)}

    def shufxor(x, sh):
        _, out = plsc.sort_key_val(keys[sh], x)
        return out
    return shufxor


def _butterfly(regs, lane, shufxor):
    stages = [(15, 1), (2, 2), (4, 4), (8, 8)]
    for mperm, bsel in stages:
        msk = (lane & bsel) == 0
        nxt = []
        for i2 in range(0, len(regs), 2):
            a, b = regs[i2], regs[i2 + 1]
            sel2 = jnp.where(msk, b, a)
            p = lax.rev(sel2, (0,)) if mperm == 15 else shufxor(sel2, mperm)
            nxt.append(jnp.where(msk, a, b) + p)
        regs = nxt
    return regs[0]


def _edge_pass_a():
    def body(xl_hbm, xr_hbm, ep_hbm, ewb_hbm, attb_hbm,
             ex_out, den_out,
             eslab, exslab, rows_s0, rows_d0, rows_s1, rows_d1, ew_sv,
             att_sv, den_v, mrg_a, mrg2d, den_sh,
             sem_s0, sem_d0, sem_s1, sem_d1):
        cid, sid, r0, cnt = _wid_rows()
        rs = jnp.minimum(r0, ROWS_TOTAL - RPT)
        off = r0 - rs
        pltpu.sync_copy(ep_hbm.at[pl.ds(rs, RPT)], eslab)
        pltpu.sync_copy(ewb_hbm, ew_sv)
        pltpu.sync_copy(attb_hbm, att_sv)
        ew_vec = ew_sv[pl.ds(0, LN)]
        att_vec = att_sv[pl.ds(0, LN)]
        _zero_1d(den_v, NPAD)
        lane = lax.iota(jnp.int32, LN)
        shufxor = _make_shufxor()

        def slabrow(i):
            return jnp.minimum(off + i, RPT - 1)

        def start_gathers(j, rs_buf, rd_buf, ss, sd):
            pltpu.async_copy(xl_hbm.at[eslab.at[j, 0]], rs_buf, ss)
            pltpu.async_copy(xr_hbm.at[eslab.at[j, 1]], rd_buf, sd)

        def wait_gather(buf, sem):
            pltpu.make_async_copy(xl_hbm.at[pl.ds(0, EB)], buf, sem).wait()

        def compute_row(i, rows_s, rows_d):
            row = slabrow(i)
            valid = i < cnt
            for v in range(EB // LN):
                d16 = eslab[row, 1, pl.ds(v * LN, LN)]
                attr16 = plsc.bitcast(
                    eslab[row, 2, pl.ds(v * LN, LN)], jnp.float32)
                regs = []
                for j in range(LN):
                    e = v * LN + j
                    u = rows_s[e, :] + rows_d[e, :] + attr16[j] * ew_vec
                    m = jnp.maximum(u, 0.2 * u)
                    regs.append(att_vec * m)
                alpha16 = _butterfly(regs, lane, shufxor)
                e16 = jnp.where(valid, jnp.exp(alpha16), 0.0)
                exslab[i, pl.ds(v * LN, LN)] = e16
                plsc.addupdate_scatter(den_v, [d16], e16)

        slots = [(rows_s0, rows_d0, sem_s0, sem_d0),
                 (rows_s1, rows_d1, sem_s1, sem_d1)]
        for b, (rs_b, rd_b, ss_b, sd_b) in enumerate(slots):
            start_gathers(slabrow(b), rs_b, rd_b, ss_b, sd_b)

        def it(k, carry):
            for b, (rs_b, rd_b, ss_b, sd_b) in enumerate(slots):
                i = 2 * k + b
                wait_gather(rs_b, ss_b)
                wait_gather(rd_b, sd_b)
                compute_row(i, rs_b, rd_b)

                @pl.when(k < NIT2 - 1)
                def _():
                    start_gathers(slabrow(i + 2), rs_b, rd_b, ss_b, sd_b)
            return carry
        lax.fori_loop(0, NIT2, it, 0)

        pltpu.sync_copy(exslab.at[pl.ds(0, ROWS_BASE)],
                        ex_out.at[pl.ds(r0, ROWS_BASE)])

        @pl.when(cnt > ROWS_BASE)
        def _():
            pltpu.sync_copy(exslab.at[ROWS_BASE],
                            ex_out.at[r0 + ROWS_BASE])

        pltpu.sync_copy(den_v, den_sh.at[sid])
        plsc.subcore_barrier()
        o2 = sid * SLICE
        pltpu.sync_copy(den_sh.at[:, pl.ds(o2, SLICE)], mrg2d)

        def addv(i, carry):
            for q in range(4):
                o3 = i * 4 * LN + q * LN
                acc = mrg2d[0, pl.ds(o3, LN)]
                for t in range(1, NS):
                    acc = acc + mrg2d[t, pl.ds(o3, LN)]
                mrg_a[pl.ds(o3, LN)] = acc
            return carry
        lax.fori_loop(0, SLICE // (4 * LN), addv, 0)
        pltpu.sync_copy(mrg_a, den_out.at[cid, pl.ds(o2, SLICE)])

    return pl.kernel(
        body,
        out_type=[jax.ShapeDtypeStruct((ROWS_TOTAL, EB), jnp.float32),
                  jax.ShapeDtypeStruct((NC, NPAD), jnp.float32)],
        mesh=_mesh(),
        scratch_types=[
            pltpu.VMEM((RPT, 3, EB), jnp.int32),
            pltpu.VMEM((RPT + 1, EB), jnp.float32),
            pltpu.VMEM((EB, WID), jnp.float32),
            pltpu.VMEM((EB, WID), jnp.float32),
            pltpu.VMEM((EB, WID), jnp.float32),
            pltpu.VMEM((EB, WID), jnp.float32),
            pltpu.VMEM((LN,), jnp.float32),
            pltpu.VMEM((LN,), jnp.float32),
            pltpu.VMEM((NPAD,), jnp.float32),
            pltpu.VMEM((SLICE,), jnp.float32),
            pltpu.VMEM((NS, SLICE), jnp.float32),
            pltpu.VMEM_SHARED((NS, NPAD), jnp.float32),
        ] + [pltpu.SemaphoreType.DMA] * 4,
        **_SC_PARAMS)


def _edge_pass_b():
    def body(xl_hbm, ep_hbm, ex_hbm, den_hbm, zeros_hbm,
             out_p,
             eslab, exslab, rows_s0, rows_s1, rows_s2, rows_s3,
             scaled0, scaled1, scaled2, scaled3,
             den_v, den_v2, out_sh,
             sem_s0, sem_s1, sem_s2, sem_s3,
             sem_o0, sem_o1, sem_o2, sem_o3):
        cid, sid, r0, cnt = _wid_rows()
        rs = jnp.minimum(r0, ROWS_TOTAL - RPT)
        off = r0 - rs
        pltpu.sync_copy(ep_hbm.at[pl.ds(rs, RPT)], eslab)
        pltpu.sync_copy(ex_hbm.at[pl.ds(rs, RPT)], exslab)
        pltpu.sync_copy(den_hbm.at[0], den_v)
        pltpu.sync_copy(den_hbm.at[1], den_v2)

        def addv(i, carry):
            for q in range(4):
                o3 = i * 4 * LN + q * LN
                den_v[pl.ds(o3, LN)] = (
                    den_v[pl.ds(o3, LN)] + den_v2[pl.ds(o3, LN)])
            return carry
        lax.fori_loop(0, NPAD // (4 * LN), addv, 0)

        o2 = sid * SLICE
        pltpu.sync_copy(zeros_hbm.at[pl.ds(o2, SLICE), :],
                        out_sh.at[pl.ds(o2, SLICE), :])
        plsc.subcore_barrier()

        def slabrow(i):
            return jnp.minimum(off + i, RPT - 1)

        def start_gather(j, rs_buf, ss):
            pltpu.async_copy(xl_hbm.at[eslab.at[j, 0]], rs_buf, ss)

        def wait_gather(buf, sem):
            pltpu.make_async_copy(xl_hbm.at[pl.ds(0, EB)], buf, sem).wait()

        def wait_scatter(buf, j, sem):
            pltpu.make_async_copy(buf, out_sh.at[eslab.at[j, 1]], sem).wait()

        def compute_scaled(i, rows_s, scaled):
            row = slabrow(i)
            valid = i < cnt
            for v in range(EB // LN):
                d16 = eslab[row, 1, pl.ds(v * LN, LN)]
                e16 = exslab[row, pl.ds(v * LN, LN)]
                den16 = plsc.load_gather(den_v, [d16])
                a16 = jnp.where(valid, e16 / (den16 + 1e-16), 0.0)
                for j in range(LN):
                    e = v * LN + j
                    scaled[e, :] = a16[j] * rows_s[e, :]

        slots = [(rows_s0, scaled0, sem_s0, sem_o0),
                 (rows_s1, scaled1, sem_s1, sem_o1),
                 (rows_s2, scaled2, sem_s2, sem_o2),
                 (rows_s3, scaled3, sem_s3, sem_o3)]
        for b, (rs_b, sc_b, ss_b, so_b) in enumerate(slots):
            start_gather(slabrow(b), rs_b, ss_b)

        def it(k, carry):
            for b, (rs_b, sc_b, ss_b, so_b) in enumerate(slots):
                i = 4 * k + b
                wait_gather(rs_b, ss_b)

                @pl.when(k > 0)
                def _():
                    wait_scatter(sc_b, slabrow(i - 4), so_b)
                compute_scaled(i, rs_b, sc_b)
                pltpu.async_copy(sc_b, out_sh.at[eslab.at[slabrow(i), 1]],
                                 so_b, add=True)

                @pl.when(k < NIT4 - 1)
                def _():
                    start_gather(slabrow(i + 4), rs_b, ss_b)
            return carry
        lax.fori_loop(0, NIT4, it, 0)
        for b, (rs_b, sc_b, ss_b, so_b) in enumerate(slots):
            wait_scatter(sc_b, slabrow(4 * NIT4 - 4 + b), so_b)
        plsc.subcore_barrier()
        pltpu.sync_copy(out_sh.at[pl.ds(o2, SLICE), :],
                        out_p.at[cid, pl.ds(o2, SLICE), :])

    return pl.kernel(
        body,
        out_type=[jax.ShapeDtypeStruct((NC, NPAD, WID), jnp.float32)],
        mesh=_mesh(),
        scratch_types=[
            pltpu.VMEM((RPT, 3, EB), jnp.int32),
            pltpu.VMEM((RPT, EB), jnp.float32),
        ] + [pltpu.VMEM((EB, WID), jnp.float32)] * 8 + [
            pltpu.VMEM((NPAD,), jnp.float32),
            pltpu.VMEM((NPAD,), jnp.float32),
            pltpu.VMEM_SHARED((NPAD, WID), jnp.float32),
        ] + [pltpu.SemaphoreType.DMA] * 8,
        **_SC_PARAMS)


def _proj_kernel(x_ref, wl_ref, bl_ref, wr_ref, br_ref, xl_ref, xr_ref):
    xv = x_ref[...]
    xl_ref[...] = jnp.dot(xv, wl_ref[...],
                          preferred_element_type=jnp.float32) + bl_ref[...]
    xr_ref[...] = jnp.dot(xv, wr_ref[...],
                          preferred_element_type=jnp.float32) + br_ref[...]


def _proj(x, wl, bl, wr, br):
    n, d = x.shape
    k = wl.shape[1]
    blk = 1024
    return pl.pallas_call(
        _proj_kernel,
        grid=(n // blk,),
        in_specs=[
            pl.BlockSpec((blk, d), lambda i: (i, 0)),
            pl.BlockSpec((d, k), lambda i: (0, 0)),
            pl.BlockSpec((1, k), lambda i: (0, 0)),
            pl.BlockSpec((d, k), lambda i: (0, 0)),
            pl.BlockSpec((1, k), lambda i: (0, 0)),
        ],
        out_specs=[pl.BlockSpec((blk, k), lambda i: (i, 0)),
                   pl.BlockSpec((blk, k), lambda i: (i, 0))],
        out_shape=[jax.ShapeDtypeStruct((n, k), jnp.float32)] * 2,
    )(x, wl, bl, wr, br)


def _merge_proj_kernel(hin, p_ref, b1_ref, wl_ref, bl_ref, wr_ref, br_ref,
                       xl_ref, xr_ref):
    p = p_ref[0][:, :hin] + p_ref[1][:, :hin]
    h = jnp.maximum(p + b1_ref[...], 0.0)
    xl_ref[...] = jnp.dot(h, wl_ref[...],
                          preferred_element_type=jnp.float32) + bl_ref[...]
    xr_ref[...] = jnp.dot(h, wr_ref[...],
                          preferred_element_type=jnp.float32) + br_ref[...]


def _merge_proj(p, b1, wl, bl, wr, br):
    _, n, wid = p.shape
    hin = wl.shape[0]
    k = wl.shape[1]
    blk = 1024
    return pl.pallas_call(
        functools.partial(_merge_proj_kernel, hin),
        grid=(n // blk,),
        in_specs=[
            pl.BlockSpec((NC, blk, wid), lambda i: (0, i, 0)),
            pl.BlockSpec((1, hin), lambda i: (0, 0)),
            pl.BlockSpec((hin, k), lambda i: (0, 0)),
            pl.BlockSpec((1, k), lambda i: (0, 0)),
            pl.BlockSpec((hin, k), lambda i: (0, 0)),
            pl.BlockSpec((1, k), lambda i: (0, 0)),
        ],
        out_specs=[pl.BlockSpec((blk, k), lambda i: (i, 0)),
                   pl.BlockSpec((blk, k), lambda i: (i, 0))],
        out_shape=[jax.ShapeDtypeStruct((n, k), jnp.float32)] * 2,
    )(p, b1, wl, bl, wr, br)


def _pool_kernel(p_ref, b2_ref, batch_ref, w3t_ref, b3_ref, y_ref,
                 s_acc, c_acc):
    k = pl.program_id(0)

    @pl.when(k == 0)
    def _():
        s_acc[...] = jnp.zeros_like(s_acc)
        c_acc[...] = jnp.zeros_like(c_acc)

    h = jnp.maximum(p_ref[0] + p_ref[1] + b2_ref[...], 0.0)
    b = batch_ref[0, 0, :]
    gi = lax.broadcasted_iota(jnp.int32, (G, PBLK), 0)
    oh = (gi == b[None, :]).astype(jnp.float32)
    s_acc[...] += jnp.dot(oh, h, preferred_element_type=jnp.float32)
    c_acc[...] += jnp.sum(oh, axis=1, keepdims=True)

    @pl.when(k == NBLK - 1)
    def _():
        g = s_acc[...] / jnp.maximum(c_acc[...], 1.0)
        y_ref[...] = jnp.dot(g, w3t_ref[...],
                             preferred_element_type=jnp.float32) + b3_ref[...]


def _pool(p, b2, batch3, w3t, b3):
    hin = p.shape[2]
    return pl.pallas_call(
        _pool_kernel,
        grid=(NBLK,),
        in_specs=[
            pl.BlockSpec((NC, PBLK, hin), lambda i: (0, i, 0)),
            pl.BlockSpec((1, hin), lambda i: (0, 0)),
            pl.BlockSpec((1, 1, PBLK), lambda i: (i, 0, 0)),
            pl.BlockSpec((hin, 1), lambda i: (0, 0)),
            pl.BlockSpec((1, 1), lambda i: (0, 0)),
        ],
        out_specs=pl.BlockSpec((G, 1), lambda i: (0, 0)),
        out_shape=jax.ShapeDtypeStruct((G, 1), jnp.float32),
        scratch_shapes=[pltpu.VMEM((G, hin), jnp.float32),
                        pltpu.VMEM((G, 1), jnp.float32)],
    )(p, b2, batch3, w3t, b3)


def kernel(x, edge_index, edge_attr, batch, lin_l_w1, lin_l_b1, lin_r_w1,
           lin_r_b1, lin_edge_w1, att1, bias1, lin_l_w2, lin_l_b2, lin_r_w2,
           lin_r_b2, lin_edge_w2, att2, bias2, w3, b3):
    n, _ = x.shape
    h1 = lin_l_w1.shape[0]
    h2 = lin_l_w2.shape[0]

    src2d = edge_index[0].reshape(ROWS_TOTAL, EB)
    dst2d = edge_index[1].reshape(ROWS_TOTAL, EB)
    attr2d = lax.bitcast_convert_type(
        edge_attr.reshape(ROWS_TOTAL, EB), jnp.int32)
    epack = jnp.stack([src2d, dst2d, attr2d], axis=1)
    x_pad = jnp.pad(x, ((0, NPAD - n), (0, 0)))
    zeros_nw = jnp.zeros((NPAD, WID), jnp.float32)

    wl1 = jnp.pad(lin_l_w1.T, ((0, 0), (0, WID - h1)))
    wr1 = jnp.pad(lin_r_w1.T, ((0, 0), (0, WID - h1)))
    bl1 = jnp.pad(lin_l_b1, (0, WID - h1))[None]
    br1 = jnp.pad(lin_r_b1, (0, WID - h1))[None]
    xl1, xr1 = _proj(x_pad, wl1, bl1, wr1, br1)
    ewb1 = jnp.pad(lin_edge_w1[:, 0], (0, WID - h1))
    attb1 = jnp.pad(att1, (0, WID - h1))
    ex1, den1 = _edge_pass_a()(xl1, xr1, epack, ewb1, attb1)
    outp1, = _edge_pass_b()(xl1, epack, ex1, den1, zeros_nw)

    xl2, xr2 = _merge_proj(outp1, bias1[None], lin_l_w2.T, lin_l_b2[None],
                           lin_r_w2.T, lin_r_b2[None])
    ewb2 = jnp.pad(lin_edge_w2[:, 0], (0, WID - h2))
    attb2 = jnp.pad(att2, (0, WID - h2))
    ex2, den2 = _edge_pass_a()(xl2, xr2, epack, ewb2, attb2)
    outp2, = _edge_pass_b()(xl2, epack, ex2, den2, zeros_nw)

    y = _pool(outp2[:, :n, :], bias2[None], batch.reshape(NBLK, 1, PBLK),
              w3.T, b3[None])
    return y
